# baseline (speedup 1.0000x reference)
import functools

import jax
import jax.numpy as jnp
from jax import lax
from jax.experimental import pallas as pl
from jax.experimental.pallas import tpu as pltpu
from jax.experimental.pallas import tpu_sc as plsc

N = 10000
E = 160000
D = 256
BN_EPS = 1e-5

NC = 2
NS = 16
L = 16
NW = NC * NS

RPT = 320
DUMMY = RPT
ACC1 = (RPT + 1) * D

CH = 8000
NCHUNK = E // CH
LCAPL = CH + 48
TRASH = CH + 32

KCH = 2048
G = 64
GPC = KCH // G
CAP = E + CH + KCH

_SC_PARAMS = pltpu.CompilerParams(needs_layout_passes=False)


def _mesh():
  return plsc.VectorSubcoreMesh(core_axis_name="c", subcore_axis_name="s",
                                num_cores=NC, num_subcores=NS)


def _wid():
  return lax.axis_index("c") * NS + lax.axis_index("s")


def _partition_body(src_hbm, dst_hbm, sl_h, dl_h, meta_h,
                    srcc, dstc, sloc, dloc, metab, sem):
  w = _wid()
  lo = w * RPT

  lov = jnp.broadcast_to(lo, (L,))
  hiv = lov + jnp.full((L,), RPT, jnp.int32)
  trashv = jnp.full((L,), TRASH, jnp.int32)
  one = jnp.full((L,), 1, jnp.int32)
  zero_i = jnp.full((L,), 0, jnp.int32)
  pad_s = jnp.full((L,), 0, jnp.int32)
  pad_d = jnp.full((L,), DUMMY, jnp.int32)
  hbase = w * CAP

  def _chunk(k, off_h):
    pltpu.sync_copy(src_hbm.at[pl.ds(k * CH, CH)], srcc)
    pltpu.sync_copy(dst_hbm.at[pl.ds(k * CH, CH)], dstc)

    def _filt(i, off):
      dv = dstc[pl.ds(i * L, L)]
      sv = srcc[pl.ds(i * L, L)]
      m = (dv >= lov) & (dv < hiv)
      mi = jnp.where(m, one, zero_i)
      excl = plsc.cumsum(mi) - mi
      offv = jnp.broadcast_to(off, (L,))
      pos = jnp.where(m, offv + excl, trashv)
      plsc.store_scatter(sloc, [pos], sv)
      plsc.store_scatter(dloc, [pos], dv - lov)
      return off + jnp.sum(mi)
    off = lax.fori_loop(0, CH // L, _filt, jnp.int32(0))

    sloc[pl.ds(off, L)] = pad_s
    dloc[pl.ds(off, L)] = pad_d
    offpad = ((off + 7) // 8) * 8

    ho = pl.multiple_of(hbase + off_h, 8)
    pltpu.sync_copy(sloc.at[pl.ds(0, CH)], sl_h.at[pl.ds(ho, CH)])
    pltpu.sync_copy(dloc.at[pl.ds(0, CH)], dl_h.at[pl.ds(ho, CH)])
    return off_h + offpad

  off_h = lax.fori_loop(0, NCHUNK, _chunk, jnp.int32(0))

  def _fill(i, _):
    srcc[pl.ds(i * L, L)] = pad_s
    dstc[pl.ds(i * L, L)] = pad_d
    return 0
  lax.fori_loop(0, KCH // L, _fill, 0)
  ho = pl.multiple_of(hbase + off_h, 8)
  pltpu.sync_copy(srcc.at[pl.ds(0, KCH)], sl_h.at[pl.ds(ho, KCH)])
  pltpu.sync_copy(dstc.at[pl.ds(0, KCH)], dl_h.at[pl.ds(ho, KCH)])

  nch = (off_h + KCH - 1) // KCH
  metab[pl.ds(0, L)] = jnp.broadcast_to(nch, (L,))
  pltpu.sync_copy(metab, meta_h.at[pl.ds(pl.multiple_of(w * L, 8), L)])


@jax.jit
def _partition(src, dst):
  f = pl.kernel(
      _partition_body,
      out_type=(
          jax.ShapeDtypeStruct((NW * CAP,), jnp.int32),
          jax.ShapeDtypeStruct((NW * CAP,), jnp.int32),
          jax.ShapeDtypeStruct((NW * L,), jnp.int32),
      ),
      mesh=_mesh(),
      compiler_params=_SC_PARAMS,
      scratch_types=[
          pltpu.VMEM((CH,), jnp.int32),
          pltpu.VMEM((CH,), jnp.int32),
          pltpu.VMEM((LCAPL,), jnp.int32),
          pltpu.VMEM((LCAPL,), jnp.int32),
          pltpu.VMEM((L,), jnp.int32),
          pltpu.SemaphoreType.DMA,
      ],
  )
  return f(src, dst)


def _segsum_body(x_hbm, sl_h, dl_h, meta_h, agg_h,
                 acc1, silist, dlist, rows0, rows1, metab, sem0, sem1):
  w = _wid()
  lo = w * RPT
  hbase = w * CAP

  zf = jnp.zeros((L,), jnp.float32)
  def _z(i, _):
    acc1[pl.ds(i * L, L)] = zf
    return 0
  lax.fori_loop(0, ACC1 // L, _z, 0)

  pltpu.sync_copy(meta_h.at[pl.ds(pl.multiple_of(w * L, 8), L)], metab)
  nch = metab[pl.ds(0, L)][0]

  def _add_group(rows, e0):
    def _q(q, _):
      dv = dlist[pl.ds(e0 + q * L, L)]
      for ll in range(L):
        r = q * L + ll
        base = dv[ll] * D
        for j in range(D // L):
          plsc.addupdate(acc1.at[pl.ds(base + j * L, L)],
                         rows[r, pl.ds(j * L, L)])
      return 0
    lax.fori_loop(0, G // L, _q, 0)

  def _chunk(kc, _):
    cbase = pl.multiple_of(hbase + kc * KCH, 8)
    pltpu.sync_copy(sl_h.at[pl.ds(cbase, KCH)], silist)
    pltpu.sync_copy(dl_h.at[pl.ds(cbase, KCH)], dlist.at[pl.ds(0, KCH)])

    def _grp(g, _):
      pltpu.async_copy(x_hbm.at[silist.at[pl.ds(g * G, G)]], rows0,
                       sem0).wait()
      _add_group(rows0, g * G)
      return 0
    lax.fori_loop(0, GPC, _grp, 0)
    return 0

  lax.fori_loop(0, nch, _chunk, 0)

  lod = pl.multiple_of(lo * D, 8)

  @pl.when(w < NW - 1)
  def _full():
    pltpu.sync_copy(acc1.at[pl.ds(0, RPT * D)],
                    agg_h.at[pl.ds(lod, RPT * D)])

  @pl.when(w == NW - 1)
  def _last():
    pltpu.sync_copy(acc1.at[pl.ds(0, (N - (NW - 1) * RPT) * D)],
                    agg_h.at[pl.ds(lod, (N - (NW - 1) * RPT) * D)])


@jax.jit
def _segsum(x, sl_h, dl_h, meta_h):
  f = pl.kernel(
      _segsum_body,
      out_type=jax.ShapeDtypeStruct((N * D,), jnp.float32),
      mesh=_mesh(),
      compiler_params=_SC_PARAMS,
      scratch_types=[
          pltpu.VMEM((ACC1,), jnp.float32),
          pltpu.VMEM((KCH,), jnp.int32),
          pltpu.VMEM((KCH + L,), jnp.int32),
          pltpu.VMEM((G, D), jnp.float32),
          pltpu.VMEM((G, D), jnp.float32),
          pltpu.VMEM((L,), jnp.int32),
          pltpu.SemaphoreType.DMA,
          pltpu.SemaphoreType.DMA,
      ],
  )
  return f(x, sl_h, dl_h, meta_h)


BR = 2000
NB = N // BR


def _mlp_body(x_ref, agg_ref, wa_ref, ba_ref, ga_ref, be_ref, wb_ref, bb_ref,
              out_ref, h1_buf, s1, s2):
  p = pl.program_id(0)
  i = pl.program_id(1)

  @pl.when(p == 0)
  def _pass0():
    @pl.when(i == 0)
    def _init():
      s1[...] = jnp.zeros_like(s1)
      s2[...] = jnp.zeros_like(s2)
    h0 = x_ref[...] + agg_ref[...]
    h1 = jnp.dot(h0, wa_ref[...], preferred_element_type=jnp.float32)
    h1 = h1 + ba_ref[...]
    h1_buf[pl.ds(i * BR, BR), :] = h1
    s1[...] += jnp.sum(h1, axis=0, keepdims=True)
    s2[...] += jnp.sum(h1 * h1, axis=0, keepdims=True)

  @pl.when(p == 1)
  def _pass1():
    mean = s1[...] * (1.0 / N)
    var = s2[...] * (1.0 / N) - mean * mean
    h1 = h1_buf[pl.ds(i * BR, BR), :]
    hn = ga_ref[...] * (h1 - mean) * lax.rsqrt(var + BN_EPS) + be_ref[...]
    hn = jnp.maximum(hn, 0.0)
    out = jnp.dot(hn, wb_ref[...], preferred_element_type=jnp.float32)
    out_ref[...] = jnp.maximum(out + bb_ref[...], 0.0)


@functools.partial(jax.jit, static_argnames=("interpret",))
def _mlp(x, agg, wa, ba, ga, be, wb, bb, interpret=False):
  row_spec = pl.BlockSpec((BR, D), lambda p, i: (i, 0))
  mat_spec = pl.BlockSpec((D, D), lambda p, i: (0, 0))
  vec_spec = pl.BlockSpec((1, D), lambda p, i: (0, 0))
  return pl.pallas_call(
      _mlp_body,
      grid=(2, NB),
      in_specs=[row_spec, row_spec, mat_spec, vec_spec, vec_spec, vec_spec,
                mat_spec, vec_spec],
      out_specs=row_spec,
      out_shape=jax.ShapeDtypeStruct((N, D), jnp.float32),
      scratch_shapes=[
          pltpu.VMEM((N, D), jnp.float32),
          pltpu.VMEM((1, D), jnp.float32),
          pltpu.VMEM((1, D), jnp.float32),
      ],
      interpret=interpret,
  )(x, agg, wa, ba.reshape(1, D), ga.reshape(1, D), be.reshape(1, D),
    wb, bb.reshape(1, D))


def kernel(x, edge_index,
           W0a, b0a, gamma0, beta0, W0b, b0b,
           W1a, b1a, gamma1, beta1, W1b, b1b,
           W2a, b2a, gamma2, beta2, W2b, b2b):
  src = edge_index[0]
  dst = edge_index[1]
  sl_h, dl_h, meta_h = _partition(src, dst)
  h = x
  for (wa, ba, ga, be, wb, bb) in (
      (W0a, b0a, gamma0, beta0, W0b, b0b),
      (W1a, b1a, gamma1, beta1, W1b, b1b),
      (W2a, b2a, gamma2, beta2, W2b, b2b)):
    agg = _segsum(h, sl_h, dl_h, meta_h).reshape(N, D)
    h = _mlp(h, agg, wa, ba, ga, be, wb, bb)
  return h


# speedup vs baseline: 1.0148x; 1.0148x over previous
import functools

import jax
import jax.numpy as jnp
from jax import lax
from jax.experimental import pallas as pl
from jax.experimental.pallas import tpu as pltpu
from jax.experimental.pallas import tpu_sc as plsc

N = 10000
E = 160000
D = 256
BN_EPS = 1e-5

NC = 2
NS = 16
L = 16
NW = NC * NS

RPT = 320
DUMMY = RPT
ACC1 = (RPT + 1) * D

CH = 8000
NCHUNK = E // CH
LCAPL = CH + 48
TRASH = CH + 32

KCH = 2048
G = 128
GPC = KCH // G
CAP = E + CH + KCH

_SC_PARAMS = pltpu.CompilerParams(needs_layout_passes=False)


def _mesh():
  return plsc.VectorSubcoreMesh(core_axis_name="c", subcore_axis_name="s",
                                num_cores=NC, num_subcores=NS)


def _wid():
  return lax.axis_index("c") * NS + lax.axis_index("s")


def _partition_body(src_hbm, dst_hbm, sl_h, dl_h, meta_h,
                    srcc, dstc, sloc, dloc, metab, sem):
  w = _wid()
  lo = w * RPT

  lov = jnp.broadcast_to(lo, (L,))
  hiv = lov + jnp.full((L,), RPT, jnp.int32)
  trashv = jnp.full((L,), TRASH, jnp.int32)
  one = jnp.full((L,), 1, jnp.int32)
  zero_i = jnp.full((L,), 0, jnp.int32)
  pad_s = jnp.full((L,), 0, jnp.int32)
  pad_d = jnp.full((L,), DUMMY, jnp.int32)
  hbase = w * CAP

  def _chunk(k, off_h):
    pltpu.sync_copy(src_hbm.at[pl.ds(k * CH, CH)], srcc)
    pltpu.sync_copy(dst_hbm.at[pl.ds(k * CH, CH)], dstc)

    def _filt(i, off):
      dv = dstc[pl.ds(i * L, L)]
      sv = srcc[pl.ds(i * L, L)]
      m = (dv >= lov) & (dv < hiv)
      mi = jnp.where(m, one, zero_i)
      excl = plsc.cumsum(mi) - mi
      offv = jnp.broadcast_to(off, (L,))
      pos = jnp.where(m, offv + excl, trashv)
      plsc.store_scatter(sloc, [pos], sv)
      plsc.store_scatter(dloc, [pos], dv - lov)
      return off + jnp.sum(mi)
    off = lax.fori_loop(0, CH // L, _filt, jnp.int32(0))

    sloc[pl.ds(off, L)] = pad_s
    dloc[pl.ds(off, L)] = pad_d
    offpad = ((off + 7) // 8) * 8

    ho = pl.multiple_of(hbase + off_h, 8)
    pltpu.sync_copy(sloc.at[pl.ds(0, CH)], sl_h.at[pl.ds(ho, CH)])
    pltpu.sync_copy(dloc.at[pl.ds(0, CH)], dl_h.at[pl.ds(ho, CH)])
    return off_h + offpad

  off_h = lax.fori_loop(0, NCHUNK, _chunk, jnp.int32(0))

  def _fill(i, _):
    srcc[pl.ds(i * L, L)] = pad_s
    dstc[pl.ds(i * L, L)] = pad_d
    return 0
  lax.fori_loop(0, KCH // L, _fill, 0)
  ho = pl.multiple_of(hbase + off_h, 8)
  pltpu.sync_copy(srcc.at[pl.ds(0, KCH)], sl_h.at[pl.ds(ho, KCH)])
  pltpu.sync_copy(dstc.at[pl.ds(0, KCH)], dl_h.at[pl.ds(ho, KCH)])

  nch = (off_h + KCH - 1) // KCH
  metab[pl.ds(0, L)] = jnp.broadcast_to(nch, (L,))
  pltpu.sync_copy(metab, meta_h.at[pl.ds(pl.multiple_of(w * L, 8), L)])


@jax.jit
def _partition(src, dst):
  f = pl.kernel(
      _partition_body,
      out_type=(
          jax.ShapeDtypeStruct((NW * CAP,), jnp.int32),
          jax.ShapeDtypeStruct((NW * CAP,), jnp.int32),
          jax.ShapeDtypeStruct((NW * L,), jnp.int32),
      ),
      mesh=_mesh(),
      compiler_params=_SC_PARAMS,
      scratch_types=[
          pltpu.VMEM((CH,), jnp.int32),
          pltpu.VMEM((CH,), jnp.int32),
          pltpu.VMEM((LCAPL,), jnp.int32),
          pltpu.VMEM((LCAPL,), jnp.int32),
          pltpu.VMEM((L,), jnp.int32),
          pltpu.SemaphoreType.DMA,
      ],
  )
  return f(src, dst)


def _segsum_body(x_hbm, sl_h, dl_h, meta_h, agg_h,
                 acc1, silist, dlist, rows0, metab, sem0):
  w = _wid()
  lo = w * RPT
  hbase = w * CAP

  zf = jnp.zeros((L,), jnp.float32)
  def _z(i, _):
    acc1[pl.ds(i * L, L)] = zf
    return 0
  lax.fori_loop(0, ACC1 // L, _z, 0)

  pltpu.sync_copy(meta_h.at[pl.ds(pl.multiple_of(w * L, 8), L)], metab)
  nch = metab[pl.ds(0, L)][0]

  def _add_group(rows, e0):
    def _q(q, _):
      dv = dlist[pl.ds(e0 + q * L, L)]
      for ll in range(L):
        r = q * L + ll
        base = dv[ll] * D
        for j in range(D // L):
          plsc.addupdate(acc1.at[pl.ds(base + j * L, L)],
                         rows[r, pl.ds(j * L, L)])
      return 0
    lax.fori_loop(0, G // L, _q, 0)

  def _chunk(kc, _):
    cbase = pl.multiple_of(hbase + kc * KCH, 8)
    pltpu.sync_copy(sl_h.at[pl.ds(cbase, KCH)], silist)
    pltpu.sync_copy(dl_h.at[pl.ds(cbase, KCH)], dlist.at[pl.ds(0, KCH)])

    def _grp(g, _):
      pltpu.async_copy(x_hbm.at[silist.at[pl.ds(g * G, G)]], rows0,
                       sem0).wait()
      _add_group(rows0, g * G)
      return 0
    lax.fori_loop(0, GPC, _grp, 0)
    return 0

  lax.fori_loop(0, nch, _chunk, 0)

  lod = pl.multiple_of(lo * D, 8)

  @pl.when(w < NW - 1)
  def _full():
    pltpu.sync_copy(acc1.at[pl.ds(0, RPT * D)],
                    agg_h.at[pl.ds(lod, RPT * D)])

  @pl.when(w == NW - 1)
  def _last():
    pltpu.sync_copy(acc1.at[pl.ds(0, (N - (NW - 1) * RPT) * D)],
                    agg_h.at[pl.ds(lod, (N - (NW - 1) * RPT) * D)])


@jax.jit
def _segsum(x, sl_h, dl_h, meta_h):
  f = pl.kernel(
      _segsum_body,
      out_type=jax.ShapeDtypeStruct((N * D,), jnp.float32),
      mesh=_mesh(),
      compiler_params=_SC_PARAMS,
      scratch_types=[
          pltpu.VMEM((ACC1,), jnp.float32),
          pltpu.VMEM((KCH,), jnp.int32),
          pltpu.VMEM((KCH + L,), jnp.int32),
          pltpu.VMEM((G, D), jnp.float32),
          pltpu.VMEM((L,), jnp.int32),
          pltpu.SemaphoreType.DMA,
      ],
  )
  return f(x, sl_h, dl_h, meta_h)


BR = 2000
NB = N // BR


def _mlp_body(x_ref, agg_ref, wa_ref, ba_ref, ga_ref, be_ref, wb_ref, bb_ref,
              out_ref, h1_buf, s1, s2):
  p = pl.program_id(0)
  i = pl.program_id(1)

  @pl.when(p == 0)
  def _pass0():
    @pl.when(i == 0)
    def _init():
      s1[...] = jnp.zeros_like(s1)
      s2[...] = jnp.zeros_like(s2)
    h0 = x_ref[...] + agg_ref[...]
    h1 = jnp.dot(h0, wa_ref[...], preferred_element_type=jnp.float32)
    h1 = h1 + ba_ref[...]
    h1_buf[pl.ds(i * BR, BR), :] = h1
    s1[...] += jnp.sum(h1, axis=0, keepdims=True)
    s2[...] += jnp.sum(h1 * h1, axis=0, keepdims=True)

  @pl.when(p == 1)
  def _pass1():
    mean = s1[...] * (1.0 / N)
    var = s2[...] * (1.0 / N) - mean * mean
    h1 = h1_buf[pl.ds(i * BR, BR), :]
    hn = ga_ref[...] * (h1 - mean) * lax.rsqrt(var + BN_EPS) + be_ref[...]
    hn = jnp.maximum(hn, 0.0)
    out = jnp.dot(hn, wb_ref[...], preferred_element_type=jnp.float32)
    out_ref[...] = jnp.maximum(out + bb_ref[...], 0.0)


@functools.partial(jax.jit, static_argnames=("interpret",))
def _mlp(x, agg, wa, ba, ga, be, wb, bb, interpret=False):
  row_spec = pl.BlockSpec((BR, D), lambda p, i: (i, 0))
  mat_spec = pl.BlockSpec((D, D), lambda p, i: (0, 0))
  vec_spec = pl.BlockSpec((1, D), lambda p, i: (0, 0))
  return pl.pallas_call(
      _mlp_body,
      grid=(2, NB),
      in_specs=[row_spec, row_spec, mat_spec, vec_spec, vec_spec, vec_spec,
                mat_spec, vec_spec],
      out_specs=row_spec,
      out_shape=jax.ShapeDtypeStruct((N, D), jnp.float32),
      scratch_shapes=[
          pltpu.VMEM((N, D), jnp.float32),
          pltpu.VMEM((1, D), jnp.float32),
          pltpu.VMEM((1, D), jnp.float32),
      ],
      interpret=interpret,
  )(x, agg, wa, ba.reshape(1, D), ga.reshape(1, D), be.reshape(1, D),
    wb, bb.reshape(1, D))


def kernel(x, edge_index,
           W0a, b0a, gamma0, beta0, W0b, b0b,
           W1a, b1a, gamma1, beta1, W1b, b1b,
           W2a, b2a, gamma2, beta2, W2b, b2b):
  src = edge_index[0]
  dst = edge_index[1]
  sl_h, dl_h, meta_h = _partition(src, dst)
  h = x
  for (wa, ba, ga, be, wb, bb) in (
      (W0a, b0a, gamma0, beta0, W0b, b0b),
      (W1a, b1a, gamma1, beta1, W1b, b1b),
      (W2a, b2a, gamma2, beta2, W2b, b2b)):
    agg = _segsum(h, sl_h, dl_h, meta_h).reshape(N, D)
    h = _mlp(h, agg, wa, ba, ga, be, wb, bb)
  return h


# speedup vs baseline: 2.5238x; 2.4870x over previous
import functools

import jax
import jax.numpy as jnp
from jax import lax
from jax.experimental import pallas as pl
from jax.experimental.pallas import tpu as pltpu
from jax.experimental.pallas import tpu_sc as plsc

N = 10000
E = 160000
D = 256
BN_EPS = 1e-5

NC = 2
NS = 16
L = 16
NW = NC * NS

RPT = 320
DUMMY = RPT
ACC1 = (RPT + 1) * D

CH = 8000
NCHUNK = E // CH
LCAPL = CH + 48
TRASH = CH + 32

KCH = 2048
G = 128
GPC = KCH // G
CAP = E + CH + KCH

_SC_PARAMS = pltpu.CompilerParams(needs_layout_passes=False)


def _mesh():
  return plsc.VectorSubcoreMesh(core_axis_name="c", subcore_axis_name="s",
                                num_cores=NC, num_subcores=NS)


def _wid():
  return lax.axis_index("c") * NS + lax.axis_index("s")


def _partition_body(src_hbm, dst_hbm, sl_h, dl_h, meta_h,
                    srcc, dstc, sloc, dloc, metab, sem):
  w = _wid()
  lo = w * RPT

  lov = jnp.broadcast_to(lo, (L,))
  hiv = lov + jnp.full((L,), RPT, jnp.int32)
  trashv = jnp.full((L,), TRASH, jnp.int32)
  one = jnp.full((L,), 1, jnp.int32)
  zero_i = jnp.full((L,), 0, jnp.int32)
  iotav = lax.iota(jnp.int32, L)
  pad_d = jnp.full((L,), DUMMY, jnp.int32)
  hbase = w * CAP

  def _chunk(k, off_h):
    pltpu.sync_copy(src_hbm.at[pl.ds(k * CH, CH)], srcc)
    pltpu.sync_copy(dst_hbm.at[pl.ds(k * CH, CH)], dstc)

    def _filt(i, off):
      dv = dstc[pl.ds(i * L, L)]
      sv = srcc[pl.ds(i * L, L)]
      m = (dv >= lov) & (dv < hiv)
      mi = jnp.where(m, one, zero_i)
      excl = plsc.cumsum(mi) - mi
      offv = jnp.broadcast_to(off, (L,))
      pos = jnp.where(m, offv + excl, trashv)
      plsc.store_scatter(sloc, [pos], sv)
      plsc.store_scatter(dloc, [pos], dv - lov)
      return off + jnp.sum(mi)
    off = lax.fori_loop(0, CH // L, _filt, jnp.int32(0))

    sloc[pl.ds(off, L)] = iotav + (k % 64) * L
    dloc[pl.ds(off, L)] = pad_d
    offpad = ((off + 7) // 8) * 8

    ho = pl.multiple_of(hbase + off_h, 8)
    pltpu.sync_copy(sloc.at[pl.ds(0, CH)], sl_h.at[pl.ds(ho, CH)])
    pltpu.sync_copy(dloc.at[pl.ds(0, CH)], dl_h.at[pl.ds(ho, CH)])
    return off_h + offpad

  off_h = lax.fori_loop(0, NCHUNK, _chunk, jnp.int32(0))

  def _fill(i, _):
    srcc[pl.ds(i * L, L)] = iotav + i * L
    dstc[pl.ds(i * L, L)] = pad_d
    return 0
  lax.fori_loop(0, G // L, _fill, 0)
  ho = pl.multiple_of(hbase + off_h, 8)
  pltpu.sync_copy(srcc.at[pl.ds(0, G)], sl_h.at[pl.ds(ho, G)])
  pltpu.sync_copy(dstc.at[pl.ds(0, G)], dl_h.at[pl.ds(ho, G)])

  ng = (off_h + G - 1) // G
  metab[pl.ds(0, L)] = jnp.broadcast_to(ng, (L,))
  pltpu.sync_copy(metab, meta_h.at[pl.ds(pl.multiple_of(w * L, 8), L)])


@jax.jit
def _partition(src, dst):
  f = pl.kernel(
      _partition_body,
      out_type=(
          jax.ShapeDtypeStruct((NW * CAP,), jnp.int32),
          jax.ShapeDtypeStruct((NW * CAP,), jnp.int32),
          jax.ShapeDtypeStruct((NW * L,), jnp.int32),
      ),
      mesh=_mesh(),
      compiler_params=_SC_PARAMS,
      scratch_types=[
          pltpu.VMEM((CH,), jnp.int32),
          pltpu.VMEM((CH,), jnp.int32),
          pltpu.VMEM((LCAPL,), jnp.int32),
          pltpu.VMEM((LCAPL,), jnp.int32),
          pltpu.VMEM((L,), jnp.int32),
          pltpu.SemaphoreType.DMA,
      ],
  )
  return f(src, dst)


def _segsum_body(x_hbm, sl_h, dl_h, meta_h, agg_h,
                 acc1, silist, dlist, rows0, metab, sem0):
  w = _wid()
  lo = w * RPT
  hbase = w * CAP

  zf = jnp.zeros((L,), jnp.float32)
  def _z(i, _):
    acc1[pl.ds(i * L, L)] = zf
    return 0
  lax.fori_loop(0, ACC1 // L, _z, 0)

  pltpu.sync_copy(meta_h.at[pl.ds(pl.multiple_of(w * L, 8), L)], metab)
  ng = metab[pl.ds(0, L)][0]
  nch = (ng + GPC - 1) // GPC

  def _add_group(rows, e0):
    @plsc.parallel_loop(0, G // L, unroll=2)
    def _q(q):
      dv = dlist[pl.ds(e0 + q * L, L)]
      for ll in range(L):
        r = q * L + ll
        base = dv[ll] * D
        for j in range(D // L):
          plsc.addupdate(acc1.at[pl.ds(base + j * L, L)],
                         rows[r, pl.ds(j * L, L)])

  def _chunk(kc, _):
    cbase = pl.multiple_of(hbase + kc * KCH, 8)
    pltpu.sync_copy(sl_h.at[pl.ds(cbase, KCH)], silist)
    pltpu.sync_copy(dl_h.at[pl.ds(cbase, KCH)], dlist.at[pl.ds(0, KCH)])

    def _grp(g, _):
      pltpu.async_copy(x_hbm.at[silist.at[pl.ds(g * G, G)]], rows0,
                       sem0).wait()
      _add_group(rows0, g * G)
      return 0
    lax.fori_loop(0, jnp.minimum(GPC, ng - kc * GPC), _grp, 0)
    return 0

  lax.fori_loop(0, nch, _chunk, 0)

  lod = pl.multiple_of(lo * D, 8)

  @pl.when(w < NW - 1)
  def _full():
    pltpu.sync_copy(acc1.at[pl.ds(0, RPT * D)],
                    agg_h.at[pl.ds(lod, RPT * D)])

  @pl.when(w == NW - 1)
  def _last():
    pltpu.sync_copy(acc1.at[pl.ds(0, (N - (NW - 1) * RPT) * D)],
                    agg_h.at[pl.ds(lod, (N - (NW - 1) * RPT) * D)])


@jax.jit
def _segsum(x, sl_h, dl_h, meta_h):
  f = pl.kernel(
      _segsum_body,
      out_type=jax.ShapeDtypeStruct((N * D,), jnp.float32),
      mesh=_mesh(),
      compiler_params=_SC_PARAMS,
      scratch_types=[
          pltpu.VMEM((ACC1,), jnp.float32),
          pltpu.VMEM((KCH,), jnp.int32),
          pltpu.VMEM((KCH + L,), jnp.int32),
          pltpu.VMEM((G, D), jnp.float32),
          pltpu.VMEM((L,), jnp.int32),
          pltpu.SemaphoreType.DMA,
      ],
  )
  return f(x, sl_h, dl_h, meta_h)


BR = 2000
NB = N // BR


def _mlp_body(x_ref, agg_ref, wa_ref, ba_ref, ga_ref, be_ref, wb_ref, bb_ref,
              out_ref, h1_buf, s1, s2):
  p = pl.program_id(0)
  i = pl.program_id(1)

  @pl.when(p == 0)
  def _pass0():
    @pl.when(i == 0)
    def _init():
      s1[...] = jnp.zeros_like(s1)
      s2[...] = jnp.zeros_like(s2)
    h0 = x_ref[...] + agg_ref[...]
    h1 = jnp.dot(h0, wa_ref[...], preferred_element_type=jnp.float32)
    h1 = h1 + ba_ref[...]
    h1_buf[pl.ds(i * BR, BR), :] = h1
    s1[...] += jnp.sum(h1, axis=0, keepdims=True)
    s2[...] += jnp.sum(h1 * h1, axis=0, keepdims=True)

  @pl.when(p == 1)
  def _pass1():
    mean = s1[...] * (1.0 / N)
    var = s2[...] * (1.0 / N) - mean * mean
    h1 = h1_buf[pl.ds(i * BR, BR), :]
    hn = ga_ref[...] * (h1 - mean) * lax.rsqrt(var + BN_EPS) + be_ref[...]
    hn = jnp.maximum(hn, 0.0)
    out = jnp.dot(hn, wb_ref[...], preferred_element_type=jnp.float32)
    out_ref[...] = jnp.maximum(out + bb_ref[...], 0.0)


@functools.partial(jax.jit, static_argnames=("interpret",))
def _mlp(x, agg, wa, ba, ga, be, wb, bb, interpret=False):
  row_spec = pl.BlockSpec((BR, D), lambda p, i: (i, 0))
  mat_spec = pl.BlockSpec((D, D), lambda p, i: (0, 0))
  vec_spec = pl.BlockSpec((1, D), lambda p, i: (0, 0))
  return pl.pallas_call(
      _mlp_body,
      grid=(2, NB),
      in_specs=[row_spec, row_spec, mat_spec, vec_spec, vec_spec, vec_spec,
                mat_spec, vec_spec],
      out_specs=row_spec,
      out_shape=jax.ShapeDtypeStruct((N, D), jnp.float32),
      scratch_shapes=[
          pltpu.VMEM((N, D), jnp.float32),
          pltpu.VMEM((1, D), jnp.float32),
          pltpu.VMEM((1, D), jnp.float32),
      ],
      interpret=interpret,
  )(x, agg, wa, ba.reshape(1, D), ga.reshape(1, D), be.reshape(1, D),
    wb, bb.reshape(1, D))


def kernel(x, edge_index,
           W0a, b0a, gamma0, beta0, W0b, b0b,
           W1a, b1a, gamma1, beta1, W1b, b1b,
           W2a, b2a, gamma2, beta2, W2b, b2b):
  src = edge_index[0]
  dst = edge_index[1]
  sl_h, dl_h, meta_h = _partition(src, dst)
  h = x
  for (wa, ba, ga, be, wb, bb) in (
      (W0a, b0a, gamma0, beta0, W0b, b0b),
      (W1a, b1a, gamma1, beta1, W1b, b1b),
      (W2a, b2a, gamma2, beta2, W2b, b2b)):
    agg = _segsum(h, sl_h, dl_h, meta_h).reshape(N, D)
    h = _mlp(h, agg, wa, ba, ga, be, wb, bb)
  return h
